# Initial kernel scaffold; baseline (speedup 1.0000x reference)
import jax
import jax.numpy as jnp
from jax.experimental import pallas as pl


def kernel(tokens, emb_table, W1, b1, W2, b2):
    raise NotImplementedError("write your pallas kernel here")


# speedup vs baseline: 7.8455x; 7.8455x over previous
import functools

import jax
import jax.numpy as jnp
from jax import lax
from jax.experimental import pallas as pl
from jax.experimental.pallas import tpu as pltpu
from jax.experimental.pallas import tpu_sc as plsc

NC = 2
NS = 16
NW = NC * NS
STREAM = 128
NBUF = 3
NPASS = 2


def _sc_pool(tokens_r, rowmap, emb_table, B, S, E):
    b_per_w = B // NW
    rows_per_w = b_per_w * S
    n_streams = rows_per_w // STREAM
    ns_pass = n_streams // NPASS
    b_pass = b_per_w // NPASS

    mesh = plsc.VectorSubcoreMesh(core_axis_name="c", subcore_axis_name="s")

    @functools.partial(
        pl.kernel,
        out_type=jax.ShapeDtypeStruct((B, E), jnp.float32),
        mesh=mesh,
        scratch_types=[
            pltpu.VMEM((n_streams, STREAM), jnp.int32),
            pltpu.VMEM((n_streams, STREAM), jnp.int32),
            pltpu.VMEM((STREAM, E), jnp.float32),
            pltpu.VMEM((STREAM, E), jnp.float32),
            pltpu.VMEM((STREAM, E), jnp.float32),
            pltpu.VMEM_SHARED((NS * b_pass, E), jnp.float32),
            pltpu.SemaphoreType.DMA,
            pltpu.SemaphoreType.DMA,
            pltpu.SemaphoreType.DMA,
        ],
    )
    def sc_pool(tokens_hbm, rowmap_hbm, table_hbm, out_hbm,
                idx_v, map_v, b0, b1, b2, pooled, g0, g1, g2):
        c = lax.axis_index("c")
        s = lax.axis_index("s")
        w = c * NS + s
        base = s * b_pass

        pltpu.sync_copy(tokens_hbm.at[w], idx_v)
        pltpu.sync_copy(rowmap_hbm.at[s], map_v)

        bufs = [b0, b1, b2]
        sems = [g0, g1, g2]
        zero = jnp.zeros((16,), jnp.float32)

        for p in range(NPASS):
            j0 = p * ns_pass

            def zbody(r, carry):
                for kk in range(E // 16):
                    b0[r, pl.ds(kk * 16, 16)] = zero
                return carry

            lax.fori_loop(0, STREAM, zbody, 0)
            for k in range(b_pass // STREAM):
                pltpu.sync_copy(b0, pooled.at[pl.ds(base + k * STREAM, STREAM)])

            for b in range(NBUF):
                pltpu.async_copy(
                    table_hbm.at[idx_v.at[j0 + b]], bufs[b], sems[b])

            def chunk(i, carry):
                for b in range(NBUF):
                    j = j0 + i * NBUF + b
                    pltpu.make_async_copy(
                        table_hbm.at[idx_v.at[j]], bufs[b], sems[b]).wait()
                    pltpu.sync_copy(bufs[b], pooled.at[map_v.at[j]], add=True)
                    nxt = j + NBUF

                    @pl.when(nxt < j0 + ns_pass)
                    def _():
                        pltpu.async_copy(
                            table_hbm.at[idx_v.at[nxt]], bufs[b], sems[b])
                return carry

            n_chunks = ns_pass // NBUF
            lax.fori_loop(0, n_chunks, chunk, 0)
            for j in range(j0 + n_chunks * NBUF, j0 + ns_pass):
                b = (j - j0) % NBUF
                pltpu.make_async_copy(
                    table_hbm.at[idx_v.at[j]], bufs[b], sems[b]).wait()
                pltpu.sync_copy(bufs[b], pooled.at[map_v.at[j]], add=True)

            pltpu.sync_copy(
                pooled.at[pl.ds(base, b_pass)],
                out_hbm.at[pl.ds(w * b_per_w + p * b_pass, b_pass)])

    return sc_pool(tokens_r, rowmap, emb_table)


def _mlp_body(x_ref, w1_ref, b1_ref, w2_ref, b2_ref, o_ref, *, inv_s):
    x = x_ref[...] * inv_s
    h = jnp.dot(x, w1_ref[...], preferred_element_type=jnp.float32)
    h = jnp.maximum(h + b1_ref[...], 0.0)
    o = jnp.dot(h, w2_ref[...], preferred_element_type=jnp.float32)
    o_ref[...] = jax.nn.sigmoid(o + b2_ref[...])


def kernel(tokens, emb_table, W1, b1, W2, b2):
    B, S = tokens.shape
    V, E = emb_table.shape
    H = W1.shape[1]
    b_per_w = B // NW
    rows_per_w = b_per_w * S
    n_streams = rows_per_w // STREAM

    tokens_r = tokens.astype(jnp.int32).reshape(NW, n_streams, STREAM)
    b_pass = b_per_w // NPASS
    t = jnp.arange(rows_per_w, dtype=jnp.int32) // S
    rowmap = (jnp.arange(NS, dtype=jnp.int32)[:, None] * b_pass
              + (t % b_pass)[None, :]).reshape(NS, n_streams, STREAM)
    pooled = _sc_pool(tokens_r, rowmap, emb_table, B, S, E)

    BT = 512
    out = pl.pallas_call(
        functools.partial(_mlp_body, inv_s=1.0 / S),
        grid=(B // BT,),
        in_specs=[
            pl.BlockSpec((BT, E), lambda i: (i, 0)),
            pl.BlockSpec((E, H), lambda i: (0, 0)),
            pl.BlockSpec((1, H), lambda i: (0, 0)),
            pl.BlockSpec((H, 1), lambda i: (0, 0)),
            pl.BlockSpec((1, 1), lambda i: (0, 0)),
        ],
        out_specs=pl.BlockSpec((BT, 1), lambda i: (i, 0)),
        out_shape=jax.ShapeDtypeStruct((B, 1), jnp.float32),
    )(pooled, W1, b1.reshape(1, H), W2, b2.reshape(1, 1))
    return out


# speedup vs baseline: 8.3821x; 1.0684x over previous
import functools

import jax
import jax.numpy as jnp
from jax import lax
from jax.experimental import pallas as pl
from jax.experimental.pallas import tpu as pltpu
from jax.experimental.pallas import tpu_sc as plsc

NC = 2
NS = 16
NW = NC * NS
STREAM = 128
NBUF = 4
NPASS = 1
NCHUNK = 2


def _make_sc_pool(Bc, S, E):
    b_per_w = Bc // NW
    rows_per_w = b_per_w * S
    n_streams = rows_per_w // STREAM
    ns_pass = n_streams // NPASS
    b_pass = b_per_w // NPASS

    mesh = plsc.VectorSubcoreMesh(core_axis_name="c", subcore_axis_name="s")

    @functools.partial(
        pl.kernel,
        out_type=jax.ShapeDtypeStruct((Bc, E), jnp.float32),
        mesh=mesh,
        scratch_types=[
            pltpu.VMEM((n_streams, STREAM), jnp.int32),
            pltpu.VMEM((n_streams, STREAM), jnp.int32),
            pltpu.VMEM((STREAM, E), jnp.float32),
            pltpu.VMEM((STREAM, E), jnp.float32),
            pltpu.VMEM((STREAM, E), jnp.float32),
            pltpu.VMEM((STREAM, E), jnp.float32),
            pltpu.VMEM_SHARED((NS * b_pass, E), jnp.float32),
            pltpu.SemaphoreType.DMA,
            pltpu.SemaphoreType.DMA,
            pltpu.SemaphoreType.DMA,
            pltpu.SemaphoreType.DMA,
        ],
    )
    def sc_pool(tokens_hbm, rowmap_hbm, table_hbm, out_hbm,
                idx_v, map_v, b0, b1, b2, b3, pooled, g0, g1, g2, g3):
        c = lax.axis_index("c")
        s = lax.axis_index("s")
        w = c * NS + s
        base = s * b_pass

        pltpu.sync_copy(tokens_hbm.at[w], idx_v)
        pltpu.sync_copy(rowmap_hbm.at[s], map_v)

        bufs = [b0, b1, b2, b3]
        sems = [g0, g1, g2, g3]
        zero = jnp.zeros((16,), jnp.float32)

        for p in range(NPASS):
            j0 = p * ns_pass

            def zbody(r, carry):
                for kk in range(E // 16):
                    b0[r, pl.ds(kk * 16, 16)] = zero
                return carry

            lax.fori_loop(0, STREAM, zbody, 0)
            for k in range(b_pass // STREAM):
                pltpu.sync_copy(b0, pooled.at[pl.ds(base + k * STREAM, STREAM)])

            for b in range(NBUF):
                pltpu.async_copy(
                    table_hbm.at[idx_v.at[j0 + b]], bufs[b], sems[b])

            def chunk(i, carry):
                for b in range(NBUF):
                    j = j0 + i * NBUF + b
                    pltpu.make_async_copy(
                        table_hbm.at[idx_v.at[j]], bufs[b], sems[b]).wait()
                    pltpu.sync_copy(bufs[b], pooled.at[map_v.at[j]], add=True)
                    nxt = j + NBUF

                    @pl.when(nxt < j0 + ns_pass)
                    def _():
                        pltpu.async_copy(
                            table_hbm.at[idx_v.at[nxt]], bufs[b], sems[b])
                return carry

            n_chunks = ns_pass // NBUF
            lax.fori_loop(0, n_chunks, chunk, 0)
            for j in range(j0 + n_chunks * NBUF, j0 + ns_pass):
                b = (j - j0) % NBUF
                pltpu.make_async_copy(
                    table_hbm.at[idx_v.at[j]], bufs[b], sems[b]).wait()
                pltpu.sync_copy(bufs[b], pooled.at[map_v.at[j]], add=True)

            pltpu.sync_copy(
                pooled.at[pl.ds(base, b_pass)],
                out_hbm.at[pl.ds(w * b_per_w + p * b_pass, b_pass)])

    return sc_pool


def _mlp_body(x_ref, w1_ref, b1_ref, w2_ref, b2_ref, o_ref, *, inv_s):
    x = x_ref[...] * inv_s
    h = jnp.dot(x, w1_ref[...], preferred_element_type=jnp.float32)
    h = jnp.maximum(h + b1_ref[...], 0.0)
    o = jnp.dot(h, w2_ref[...], preferred_element_type=jnp.float32)
    o_ref[...] = jax.nn.sigmoid(o + b2_ref[...])


def kernel(tokens, emb_table, W1, b1, W2, b2):
    B, S = tokens.shape
    V, E = emb_table.shape
    H = W1.shape[1]
    Bc = B // NCHUNK
    b_per_w = Bc // NW
    rows_per_w = b_per_w * S
    n_streams = rows_per_w // STREAM
    b_pass = b_per_w // NPASS

    tokens_r = tokens.astype(jnp.int32).reshape(NCHUNK, NW, n_streams, STREAM)
    t = jnp.arange(rows_per_w, dtype=jnp.int32) // S
    rowmap = (jnp.arange(NS, dtype=jnp.int32)[:, None] * b_pass
              + (t % b_pass)[None, :]).reshape(NS, n_streams, STREAM)

    sc_pool = _make_sc_pool(Bc, S, E)
    b1r = b1.reshape(1, H)
    b2r = b2.reshape(1, 1)

    BT = 512
    mlp = pl.pallas_call(
        functools.partial(_mlp_body, inv_s=1.0 / S),
        grid=(Bc // BT,),
        in_specs=[
            pl.BlockSpec((BT, E), lambda i: (i, 0)),
            pl.BlockSpec((E, H), lambda i: (0, 0)),
            pl.BlockSpec((1, H), lambda i: (0, 0)),
            pl.BlockSpec((H, 1), lambda i: (0, 0)),
            pl.BlockSpec((1, 1), lambda i: (0, 0)),
        ],
        out_specs=pl.BlockSpec((BT, 1), lambda i: (i, 0)),
        out_shape=jax.ShapeDtypeStruct((Bc, 1), jnp.float32),
    )

    outs = []
    for ck in range(NCHUNK):
        pooled = sc_pool(tokens_r[ck], rowmap, emb_table)
        outs.append(mlp(pooled, W1, b1r, W2, b2r))
    return jnp.concatenate(outs, axis=0)


# speedup vs baseline: 8.5891x; 1.0247x over previous
import functools

import jax
import jax.numpy as jnp
from jax import lax
from jax.experimental import pallas as pl
from jax.experimental.pallas import tpu as pltpu
from jax.experimental.pallas import tpu_sc as plsc

NC = 2
NS = 16
NW = NC * NS
STREAM = 128
NBUF = 4
NPASS = 1
NCHUNK = 4


def _make_sc_pool(Bc, S, E):
    b_per_w = Bc // NW
    rows_per_w = b_per_w * S
    n_streams = rows_per_w // STREAM
    ns_pass = n_streams // NPASS
    b_pass = b_per_w // NPASS

    mesh = plsc.VectorSubcoreMesh(core_axis_name="c", subcore_axis_name="s")

    @functools.partial(
        pl.kernel,
        out_type=jax.ShapeDtypeStruct((Bc, E), jnp.float32),
        mesh=mesh,
        scratch_types=[
            pltpu.VMEM((n_streams, STREAM), jnp.int32),
            pltpu.VMEM((n_streams, STREAM), jnp.int32),
            pltpu.VMEM((STREAM, E), jnp.float32),
            pltpu.VMEM((STREAM, E), jnp.float32),
            pltpu.VMEM((STREAM, E), jnp.float32),
            pltpu.VMEM((STREAM, E), jnp.float32),
            pltpu.VMEM_SHARED((NS * b_pass, E), jnp.float32),
            pltpu.SemaphoreType.DMA,
            pltpu.SemaphoreType.DMA,
            pltpu.SemaphoreType.DMA,
            pltpu.SemaphoreType.DMA,
        ],
    )
    def sc_pool(tokens_hbm, rowmap_hbm, table_hbm, out_hbm,
                idx_v, map_v, b0, b1, b2, b3, pooled, g0, g1, g2, g3):
        c = lax.axis_index("c")
        s = lax.axis_index("s")
        w = c * NS + s
        base = s * b_pass

        pltpu.sync_copy(tokens_hbm.at[w], idx_v)
        pltpu.sync_copy(rowmap_hbm.at[s], map_v)

        bufs = [b0, b1, b2, b3]
        sems = [g0, g1, g2, g3]
        zero = jnp.zeros((16,), jnp.float32)

        for p in range(NPASS):
            j0 = p * ns_pass

            def zbody(r, carry):
                for kk in range(E // 16):
                    b0[r, pl.ds(kk * 16, 16)] = zero
                return carry

            lax.fori_loop(0, STREAM, zbody, 0)
            for k in range(b_pass // STREAM):
                pltpu.sync_copy(b0, pooled.at[pl.ds(base + k * STREAM, STREAM)])

            for b in range(NBUF):
                pltpu.async_copy(
                    table_hbm.at[idx_v.at[j0 + b]], bufs[b], sems[b])

            def chunk(i, carry):
                for b in range(NBUF):
                    j = j0 + i * NBUF + b
                    pltpu.make_async_copy(
                        table_hbm.at[idx_v.at[j]], bufs[b], sems[b]).wait()
                    pltpu.sync_copy(bufs[b], pooled.at[map_v.at[j]], add=True)
                    nxt = j + NBUF

                    @pl.when(nxt < j0 + ns_pass)
                    def _():
                        pltpu.async_copy(
                            table_hbm.at[idx_v.at[nxt]], bufs[b], sems[b])
                return carry

            n_chunks = ns_pass // NBUF
            lax.fori_loop(0, n_chunks, chunk, 0)
            for j in range(j0 + n_chunks * NBUF, j0 + ns_pass):
                b = (j - j0) % NBUF
                pltpu.make_async_copy(
                    table_hbm.at[idx_v.at[j]], bufs[b], sems[b]).wait()
                pltpu.sync_copy(bufs[b], pooled.at[map_v.at[j]], add=True)

            pltpu.sync_copy(
                pooled.at[pl.ds(base, b_pass)],
                out_hbm.at[pl.ds(w * b_per_w + p * b_pass, b_pass)])

    return sc_pool


def _mlp_body(x_ref, w1_ref, b1_ref, w2_ref, b2_ref, o_ref, *, inv_s):
    x = x_ref[...] * inv_s
    h = jnp.dot(x, w1_ref[...], preferred_element_type=jnp.float32)
    h = jnp.maximum(h + b1_ref[...], 0.0)
    o = jnp.dot(h, w2_ref[...], preferred_element_type=jnp.float32)
    o_ref[...] = jax.nn.sigmoid(o + b2_ref[...])


def kernel(tokens, emb_table, W1, b1, W2, b2):
    B, S = tokens.shape
    V, E = emb_table.shape
    H = W1.shape[1]
    Bc = B // NCHUNK
    b_per_w = Bc // NW
    rows_per_w = b_per_w * S
    n_streams = rows_per_w // STREAM
    b_pass = b_per_w // NPASS

    tokens = tokens.astype(jnp.int32)
    t = jnp.arange(rows_per_w, dtype=jnp.int32) // S
    rowmap = (jnp.arange(NS, dtype=jnp.int32)[:, None] * b_pass
              + (t % b_pass)[None, :]).reshape(NS, n_streams, STREAM)

    sc_pool = _make_sc_pool(Bc, S, E)
    b1r = b1.reshape(1, H)
    b2r = b2.reshape(1, 1)

    BT = 512
    mlp = pl.pallas_call(
        functools.partial(_mlp_body, inv_s=1.0 / S),
        grid=(Bc // BT,),
        in_specs=[
            pl.BlockSpec((BT, E), lambda i: (i, 0)),
            pl.BlockSpec((E, H), lambda i: (0, 0)),
            pl.BlockSpec((1, H), lambda i: (0, 0)),
            pl.BlockSpec((H, 1), lambda i: (0, 0)),
            pl.BlockSpec((1, 1), lambda i: (0, 0)),
        ],
        out_specs=pl.BlockSpec((BT, 1), lambda i: (i, 0)),
        out_shape=jax.ShapeDtypeStruct((Bc, 1), jnp.float32),
    )

    outs = []
    for ck in range(NCHUNK):
        tokens_c = lax.slice_in_dim(tokens, ck * Bc, (ck + 1) * Bc, axis=0)
        tokens_r = tokens_c.reshape(NW, n_streams, STREAM)
        pooled = sc_pool(tokens_r, rowmap, emb_table)
        outs.append(mlp(pooled, W1, b1r, W2, b2r))
    return jnp.concatenate(outs, axis=0)


# speedup vs baseline: 8.7901x; 1.0234x over previous
import functools

import jax
import jax.numpy as jnp
from jax import lax
from jax.experimental import pallas as pl
from jax.experimental.pallas import tpu as pltpu
from jax.experimental.pallas import tpu_sc as plsc

NC = 2
NS = 16
NW = NC * NS
STREAM = 128
NBUF = 4
NPASS = 1
NCHUNK = 4


def _make_sc_pool(Bc, S, E):
    b_per_w = Bc // NW
    rows_per_w = b_per_w * S
    n_streams = rows_per_w // STREAM
    ns_pass = n_streams // NPASS
    b_pass = b_per_w // NPASS

    mesh = plsc.VectorSubcoreMesh(core_axis_name="c", subcore_axis_name="s")

    @functools.partial(
        pl.kernel,
        out_type=jax.ShapeDtypeStruct((Bc, E), jnp.float32),
        mesh=mesh,
        scratch_types=[
            pltpu.VMEM((n_streams, STREAM), jnp.int32),
            pltpu.VMEM((n_streams, STREAM), jnp.int32),
            pltpu.VMEM((STREAM, E), jnp.float32),
            pltpu.VMEM((STREAM, E), jnp.float32),
            pltpu.VMEM((STREAM, E), jnp.float32),
            pltpu.VMEM((STREAM, E), jnp.float32),
            pltpu.VMEM_SHARED((NS * b_pass, E), jnp.float32),
            pltpu.SemaphoreType.DMA,
            pltpu.SemaphoreType.DMA,
            pltpu.SemaphoreType.DMA,
            pltpu.SemaphoreType.DMA,
            pltpu.SemaphoreType.DMA,
            pltpu.SemaphoreType.DMA,
            pltpu.SemaphoreType.DMA,
            pltpu.SemaphoreType.DMA,
        ],
    )
    def sc_pool(tokens_hbm, rowmap_hbm, table_hbm, out_hbm,
                idx_v, map_v, b0, b1, b2, b3, pooled,
                g0, g1, g2, g3, s0, s1, s2, s3):
        c = lax.axis_index("c")
        s = lax.axis_index("s")
        w = c * NS + s
        base = s * b_pass

        pltpu.sync_copy(tokens_hbm.at[w], idx_v)
        pltpu.sync_copy(rowmap_hbm.at[s], map_v)

        bufs = [b0, b1, b2, b3]
        gsems = [g0, g1, g2, g3]
        ssems = [s0, s1, s2, s3]
        zero = jnp.zeros((16,), jnp.float32)

        for p in range(NPASS):
            j0 = p * ns_pass

            def zbody(r, carry):
                for kk in range(E // 16):
                    b0[r, pl.ds(kk * 16, 16)] = zero
                return carry

            lax.fori_loop(0, STREAM, zbody, 0)
            for k in range(b_pass // STREAM):
                pltpu.sync_copy(b0, pooled.at[pl.ds(base + k * STREAM, STREAM)])

            for b in range(NBUF):
                pltpu.async_copy(
                    table_hbm.at[idx_v.at[j0 + b]], bufs[b], gsems[b])

            def step(j, b, first):
                pltpu.make_async_copy(
                    table_hbm.at[idx_v.at[j]], bufs[b], gsems[b]).wait()
                pltpu.async_copy(
                    bufs[b], pooled.at[map_v.at[j]], ssems[b], add=True)
                if not first:
                    bp = (b - 1) % NBUF
                    pltpu.make_async_copy(
                        bufs[bp], pooled.at[map_v.at[j]], ssems[bp]).wait()
                    nxt = j + NBUF - 1

                    @pl.when(nxt < j0 + ns_pass)
                    def _():
                        pltpu.async_copy(
                            table_hbm.at[idx_v.at[nxt]], bufs[bp], gsems[bp])

            for b in range(NBUF):
                step(j0 + b, b, first=(b == 0))

            def chunk(i, carry):
                for b in range(NBUF):
                    step(j0 + i * NBUF + b, b, first=False)
                return carry

            n_chunks = ns_pass // NBUF
            lax.fori_loop(1, n_chunks, chunk, 0)
            last = (ns_pass - 1) % NBUF
            pltpu.make_async_copy(
                bufs[last], pooled.at[map_v.at[j0]], ssems[last]).wait()

            pltpu.sync_copy(
                pooled.at[pl.ds(base, b_pass)],
                out_hbm.at[pl.ds(w * b_per_w + p * b_pass, b_pass)])

    return sc_pool


def _mlp_body(x_ref, w1_ref, b1_ref, w2_ref, b2_ref, o_ref, *, inv_s):
    x = x_ref[...] * inv_s
    h = jnp.dot(x, w1_ref[...], preferred_element_type=jnp.float32)
    h = jnp.maximum(h + b1_ref[...], 0.0)
    o = jnp.dot(h, w2_ref[...], preferred_element_type=jnp.float32)
    o_ref[...] = jax.nn.sigmoid(o + b2_ref[...])


def kernel(tokens, emb_table, W1, b1, W2, b2):
    B, S = tokens.shape
    V, E = emb_table.shape
    H = W1.shape[1]
    Bc = B // NCHUNK
    b_per_w = Bc // NW
    rows_per_w = b_per_w * S
    n_streams = rows_per_w // STREAM
    b_pass = b_per_w // NPASS

    tokens = tokens.astype(jnp.int32)
    t = jnp.arange(rows_per_w, dtype=jnp.int32) // S
    rowmap = (jnp.arange(NS, dtype=jnp.int32)[:, None] * b_pass
              + (t % b_pass)[None, :]).reshape(NS, n_streams, STREAM)

    sc_pool = _make_sc_pool(Bc, S, E)
    b1r = b1.reshape(1, H)
    b2r = b2.reshape(1, 1)

    BT = 512
    mlp = pl.pallas_call(
        functools.partial(_mlp_body, inv_s=1.0 / S),
        grid=(Bc // BT,),
        in_specs=[
            pl.BlockSpec((BT, E), lambda i: (i, 0)),
            pl.BlockSpec((E, H), lambda i: (0, 0)),
            pl.BlockSpec((1, H), lambda i: (0, 0)),
            pl.BlockSpec((H, 1), lambda i: (0, 0)),
            pl.BlockSpec((1, 1), lambda i: (0, 0)),
        ],
        out_specs=pl.BlockSpec((BT, 1), lambda i: (i, 0)),
        out_shape=jax.ShapeDtypeStruct((Bc, 1), jnp.float32),
    )

    outs = []
    for ck in range(NCHUNK):
        tokens_c = lax.slice_in_dim(tokens, ck * Bc, (ck + 1) * Bc, axis=0)
        tokens_r = tokens_c.reshape(NW, n_streams, STREAM)
        pooled = sc_pool(tokens_r, rowmap, emb_table)
        outs.append(mlp(pooled, W1, b1r, W2, b2r))
    return jnp.concatenate(outs, axis=0)
